# baseline (device time: 791651 ns/iter reference)
import jax
import jax.numpy as jnp
from jax import lax
from jax.experimental import pallas as pl
from jax.experimental.pallas import tpu as pltpu

N_DEV = 32
N_SLOTS = 4


def kernel(x, w_mat):
    m_global, k_per = x.shape
    _, n = w_mat.shape
    m_per = m_global // N_DEV

    def body(x_ref, w_ref, out_ref, wbf_ref, comm_ref, send_sems, recv_sems):
        my = lax.axis_index("i")
        left = lax.rem(my + N_DEV - 1, N_DEV)
        right = lax.rem(my + 1, N_DEV)

        barrier = pltpu.get_barrier_semaphore()
        for nbr in (left, right):
            pl.semaphore_signal(
                barrier, inc=1, device_id=(nbr,),
                device_id_type=pl.DeviceIdType.MESH,
            )
        pl.semaphore_wait(barrier, 2)

        wbf_ref[...] = w_ref[...].astype(jnp.bfloat16)

        def block(offset):
            c = lax.rem(my + (2 * N_DEV - offset), N_DEV)
            xa = x_ref[pl.ds(c * m_per, m_per), :].astype(jnp.bfloat16)
            return jnp.dot(xa, wbf_ref[...], preferred_element_type=jnp.float32)

        comm_ref[0, :, :] = block(1).astype(jnp.bfloat16)

        for t in range(N_DEV - 1):
            s = t % N_SLOTS
            r = (t + 1) % N_SLOTS
            rdma = pltpu.make_async_remote_copy(
                src_ref=comm_ref.at[s],
                dst_ref=comm_ref.at[r],
                send_sem=send_sems.at[s],
                recv_sem=recv_sems.at[r],
                device_id=(right,),
                device_id_type=pl.DeviceIdType.MESH,
            )
            rdma.start()
            blk = block(t + 2)
            rdma.wait()
            acc = comm_ref[r, :, :].astype(jnp.float32) + blk
            if t < N_DEV - 2:
                comm_ref[r, :, :] = acc.astype(jnp.bfloat16)
            else:
                out_ref[...] = jnp.maximum(acc, 0.0)

    return pl.pallas_call(
        body,
        out_shape=jax.ShapeDtypeStruct((m_per, n), jnp.float32),
        in_specs=[
            pl.BlockSpec(memory_space=pltpu.VMEM),
            pl.BlockSpec(memory_space=pltpu.VMEM),
        ],
        out_specs=pl.BlockSpec(memory_space=pltpu.VMEM),
        scratch_shapes=[
            pltpu.VMEM((k_per, n), jnp.bfloat16),
            pltpu.VMEM((N_SLOTS, m_per, n), jnp.bfloat16),
            pltpu.SemaphoreType.DMA((N_SLOTS,)),
            pltpu.SemaphoreType.DMA((N_SLOTS,)),
        ],
        compiler_params=pltpu.CompilerParams(collective_id=0),
    )(x, w_mat)


# device time: 774955 ns/iter; 1.0215x vs baseline; 1.0215x over previous
import jax
import jax.numpy as jnp
from jax import lax
from jax.experimental import pallas as pl
from jax.experimental.pallas import tpu as pltpu

N_DEV = 32
N_SLOTS = 4


def kernel(x, w_mat):
    m_global, k_per = x.shape
    _, n = w_mat.shape
    m_per = m_global // N_DEV
    nh = n // 2

    def body(x_ref, w_ref, out_ref, wbf_ref, commr_ref, comml_ref,
             send_r, recv_r, send_l, recv_l):
        my = lax.axis_index("i")
        left = lax.rem(my + N_DEV - 1, N_DEV)
        right = lax.rem(my + 1, N_DEV)

        barrier = pltpu.get_barrier_semaphore()
        for nbr in (left, right):
            pl.semaphore_signal(
                barrier, inc=1, device_id=(nbr,),
                device_id_type=pl.DeviceIdType.MESH,
            )
        pl.semaphore_wait(barrier, 2)

        wbf_ref[...] = w_ref[...].astype(jnp.bfloat16)

        def block_r(offset):
            c = lax.rem(my + (2 * N_DEV - offset), N_DEV)
            xa = x_ref[pl.ds(c * m_per, m_per), :].astype(jnp.bfloat16)
            return jnp.dot(xa, wbf_ref[:, :nh],
                           preferred_element_type=jnp.float32)

        def block_l(offset):
            c = lax.rem(my + offset, N_DEV)
            xa = x_ref[pl.ds(c * m_per, m_per), :].astype(jnp.bfloat16)
            return jnp.dot(xa, wbf_ref[:, nh:],
                           preferred_element_type=jnp.float32)

        commr_ref[0, :, :] = block_r(1).astype(jnp.bfloat16)
        comml_ref[0, :, :] = block_l(1).astype(jnp.bfloat16)

        for t in range(N_DEV - 1):
            s = t % N_SLOTS
            r = (t + 1) % N_SLOTS
            rdma_r = pltpu.make_async_remote_copy(
                src_ref=commr_ref.at[s],
                dst_ref=commr_ref.at[r],
                send_sem=send_r.at[s],
                recv_sem=recv_r.at[r],
                device_id=(right,),
                device_id_type=pl.DeviceIdType.MESH,
            )
            rdma_l = pltpu.make_async_remote_copy(
                src_ref=comml_ref.at[s],
                dst_ref=comml_ref.at[r],
                send_sem=send_l.at[s],
                recv_sem=recv_l.at[r],
                device_id=(left,),
                device_id_type=pl.DeviceIdType.MESH,
            )
            rdma_r.start()
            rdma_l.start()
            blk_r = block_r(t + 2)
            blk_l = block_l(t + 2)
            rdma_r.wait()
            acc_r = commr_ref[r, :, :].astype(jnp.float32) + blk_r
            if t < N_DEV - 2:
                commr_ref[r, :, :] = acc_r.astype(jnp.bfloat16)
            else:
                out_ref[:, :nh] = jnp.maximum(acc_r, 0.0)
            rdma_l.wait()
            acc_l = comml_ref[r, :, :].astype(jnp.float32) + blk_l
            if t < N_DEV - 2:
                comml_ref[r, :, :] = acc_l.astype(jnp.bfloat16)
            else:
                out_ref[:, nh:] = jnp.maximum(acc_l, 0.0)

    return pl.pallas_call(
        body,
        out_shape=jax.ShapeDtypeStruct((m_per, n), jnp.float32),
        in_specs=[
            pl.BlockSpec(memory_space=pltpu.VMEM),
            pl.BlockSpec(memory_space=pltpu.VMEM),
        ],
        out_specs=pl.BlockSpec(memory_space=pltpu.VMEM),
        scratch_shapes=[
            pltpu.VMEM((k_per, n), jnp.bfloat16),
            pltpu.VMEM((N_SLOTS, m_per, nh), jnp.bfloat16),
            pltpu.VMEM((N_SLOTS, m_per, nh), jnp.bfloat16),
            pltpu.SemaphoreType.DMA((N_SLOTS,)),
            pltpu.SemaphoreType.DMA((N_SLOTS,)),
            pltpu.SemaphoreType.DMA((N_SLOTS,)),
            pltpu.SemaphoreType.DMA((N_SLOTS,)),
        ],
        compiler_params=pltpu.CompilerParams(collective_id=0),
    )(x, w_mat)


# device time: 442682 ns/iter; 1.7883x vs baseline; 1.7506x over previous
import jax
import jax.numpy as jnp
from jax import lax
from jax.experimental import pallas as pl
from jax.experimental.pallas import tpu as pltpu

N_DEV = 32
N_SLOTS = 4

_PLANE_SNAKE = [(0, 0), (1, 0), (1, 1), (0, 1), (0, 2), (1, 2), (1, 3), (0, 3)]
_LOGICAL_OF = {}
for _z in range(4):
    for _xy in _PLANE_SNAKE:
        _LOGICAL_OF[(_xy[0], _xy[1], _z)] = len(_LOGICAL_OF)

_CYC_COORDS = []
for _z in range(4):
    _ys = range(4) if _z % 2 == 0 else range(3, -1, -1)
    _CYC_COORDS += [(0, _y, _z) for _y in _ys]
for _z in range(3, -1, -1):
    _ys = range(4) if _z % 2 == 1 else range(3, -1, -1)
    _CYC_COORDS += [(1, _y, _z) for _y in _ys]
assert len(set(_CYC_COORDS)) == N_DEV
for _a, _b in zip(_CYC_COORDS, _CYC_COORDS[1:] + _CYC_COORDS[:1]):
    assert sum(abs(_i - _j) for _i, _j in zip(_a, _b)) == 1, (_a, _b)

_CYCLE = [_LOGICAL_OF[c] for c in _CYC_COORDS]
_INV = [0] * N_DEV
for _p, _l in enumerate(_CYCLE):
    _INV[_l] = _p


def kernel(x, w_mat):
    m_global, k_per = x.shape
    _, n = w_mat.shape
    m_per = m_global // N_DEV
    nh = n // 2

    cyc = jnp.asarray(_CYCLE, dtype=jnp.int32)
    inv = jnp.asarray(_INV, dtype=jnp.int32)
    my = lax.axis_index("i")
    j = inv[my]
    nxt = cyc[(j + 1) % N_DEV]
    prv = cyc[(j - 1) % N_DEV]
    i_arr = jnp.arange(N_DEV, dtype=jnp.int32)
    sched_r = cyc[(j - 1 - i_arr) % N_DEV]
    sched_l = cyc[(j + 1 + i_arr) % N_DEV]
    meta = jnp.stack([nxt, prv]).astype(jnp.int32)

    def body(meta_ref, schr_ref, schl_ref, x_ref, w_ref, out_ref,
             wbf_ref, commr_ref, comml_ref, send_r, recv_r, send_l, recv_l):
        nxt_ = meta_ref[0]
        prv_ = meta_ref[1]

        barrier = pltpu.get_barrier_semaphore()
        for nbr in (nxt_, prv_):
            pl.semaphore_signal(
                barrier, inc=1, device_id=(nbr,),
                device_id_type=pl.DeviceIdType.MESH,
            )
        pl.semaphore_wait(barrier, 2)

        wbf_ref[...] = w_ref[...].astype(jnp.bfloat16)

        def block_r(i):
            c = schr_ref[i]
            xa = x_ref[pl.ds(c * m_per, m_per), :].astype(jnp.bfloat16)
            return jnp.dot(xa, wbf_ref[:, :nh],
                           preferred_element_type=jnp.float32)

        def block_l(i):
            c = schl_ref[i]
            xa = x_ref[pl.ds(c * m_per, m_per), :].astype(jnp.bfloat16)
            return jnp.dot(xa, wbf_ref[:, nh:],
                           preferred_element_type=jnp.float32)

        commr_ref[0, :, :] = block_r(0).astype(jnp.bfloat16)
        comml_ref[0, :, :] = block_l(0).astype(jnp.bfloat16)

        for t in range(N_DEV - 1):
            s = t % N_SLOTS
            r = (t + 1) % N_SLOTS
            rdma_r = pltpu.make_async_remote_copy(
                src_ref=commr_ref.at[s],
                dst_ref=commr_ref.at[r],
                send_sem=send_r.at[s],
                recv_sem=recv_r.at[r],
                device_id=(nxt_,),
                device_id_type=pl.DeviceIdType.MESH,
            )
            rdma_l = pltpu.make_async_remote_copy(
                src_ref=comml_ref.at[s],
                dst_ref=comml_ref.at[r],
                send_sem=send_l.at[s],
                recv_sem=recv_l.at[r],
                device_id=(prv_,),
                device_id_type=pl.DeviceIdType.MESH,
            )
            rdma_r.start()
            rdma_l.start()
            blk_r = block_r(t + 1)
            blk_l = block_l(t + 1)
            rdma_r.wait()
            acc_r = commr_ref[r, :, :].astype(jnp.float32) + blk_r
            if t < N_DEV - 2:
                commr_ref[r, :, :] = acc_r.astype(jnp.bfloat16)
            else:
                out_ref[:, :nh] = jnp.maximum(acc_r, 0.0)
            rdma_l.wait()
            acc_l = comml_ref[r, :, :].astype(jnp.float32) + blk_l
            if t < N_DEV - 2:
                comml_ref[r, :, :] = acc_l.astype(jnp.bfloat16)
            else:
                out_ref[:, nh:] = jnp.maximum(acc_l, 0.0)

    return pl.pallas_call(
        body,
        out_shape=jax.ShapeDtypeStruct((m_per, n), jnp.float32),
        in_specs=[
            pl.BlockSpec(memory_space=pltpu.SMEM),
            pl.BlockSpec(memory_space=pltpu.SMEM),
            pl.BlockSpec(memory_space=pltpu.SMEM),
            pl.BlockSpec(memory_space=pltpu.VMEM),
            pl.BlockSpec(memory_space=pltpu.VMEM),
        ],
        out_specs=pl.BlockSpec(memory_space=pltpu.VMEM),
        scratch_shapes=[
            pltpu.VMEM((k_per, n), jnp.bfloat16),
            pltpu.VMEM((N_SLOTS, m_per, nh), jnp.bfloat16),
            pltpu.VMEM((N_SLOTS, m_per, nh), jnp.bfloat16),
            pltpu.SemaphoreType.DMA((N_SLOTS,)),
            pltpu.SemaphoreType.DMA((N_SLOTS,)),
            pltpu.SemaphoreType.DMA((N_SLOTS,)),
            pltpu.SemaphoreType.DMA((N_SLOTS,)),
        ],
        compiler_params=pltpu.CompilerParams(collective_id=0),
    )(meta, sched_r, sched_l, x, w_mat)


# device time: 368401 ns/iter; 2.1489x vs baseline; 1.2016x over previous
import jax
import jax.numpy as jnp
from jax import lax
from jax.experimental import pallas as pl
from jax.experimental.pallas import tpu as pltpu

N_DEV = 32
N_SLOTS = 4

_PLANE_SNAKE = [(0, 0), (1, 0), (1, 1), (0, 1), (0, 2), (1, 2), (1, 3), (0, 3)]
_LOGICAL_OF = {}
for _z in range(4):
    for _xy in _PLANE_SNAKE:
        _LOGICAL_OF[(_xy[0], _xy[1], _z)] = len(_LOGICAL_OF)

_CYC_COORDS = []
for _z in range(4):
    _ys = range(4) if _z % 2 == 0 else range(3, -1, -1)
    _CYC_COORDS += [(0, _y, _z) for _y in _ys]
for _z in range(3, -1, -1):
    _ys = range(4) if _z % 2 == 1 else range(3, -1, -1)
    _CYC_COORDS += [(1, _y, _z) for _y in _ys]
assert len(set(_CYC_COORDS)) == N_DEV
for _a, _b in zip(_CYC_COORDS, _CYC_COORDS[1:] + _CYC_COORDS[:1]):
    assert sum(abs(_i - _j) for _i, _j in zip(_a, _b)) == 1, (_a, _b)

_CYCLE = [_LOGICAL_OF[c] for c in _CYC_COORDS]
_INV = [0] * N_DEV
for _p, _l in enumerate(_CYCLE):
    _INV[_l] = _p


def kernel(x, w_mat):
    m_global, k_per = x.shape
    _, n = w_mat.shape
    m_per = m_global // N_DEV
    nh = n // 2
    nq = n // 4

    cyc = jnp.asarray(_CYCLE, dtype=jnp.int32)
    inv = jnp.asarray(_INV, dtype=jnp.int32)
    my = lax.axis_index("i")
    j = inv[my]
    nxt = cyc[(j + 1) % N_DEV]
    prv = cyc[(j - 1) % N_DEV]
    i_arr = jnp.arange(N_DEV, dtype=jnp.int32)
    sched_r = cyc[(j - 1 - i_arr) % N_DEV]
    sched_l = cyc[(j + 1 + i_arr) % N_DEV]
    meta = jnp.stack([nxt, prv]).astype(jnp.int32)

    def body(meta_ref, schr_ref, schl_ref, x_ref, w_ref, out_ref, wbf_ref,
             comm_r0, comm_r1, comm_l0, comm_l1,
             send_r0, recv_r0, send_r1, recv_r1,
             send_l0, recv_l0, send_l1, recv_l1):
        nxt_ = meta_ref[0]
        prv_ = meta_ref[1]

        barrier = pltpu.get_barrier_semaphore()
        for nbr in (nxt_, prv_):
            pl.semaphore_signal(
                barrier, inc=1, device_id=(nbr,),
                device_id_type=pl.DeviceIdType.MESH,
            )
        pl.semaphore_wait(barrier, 2)

        wbf_ref[...] = w_ref[...].astype(jnp.bfloat16)

        def block_r(i):
            c = schr_ref[i]
            xa = x_ref[pl.ds(c * m_per, m_per), :].astype(jnp.bfloat16)
            return jnp.dot(xa, wbf_ref[:, :nh],
                           preferred_element_type=jnp.float32)

        def block_l(i):
            c = schl_ref[i]
            xa = x_ref[pl.ds(c * m_per, m_per), :].astype(jnp.bfloat16)
            return jnp.dot(xa, wbf_ref[:, nh:],
                           preferred_element_type=jnp.float32)

        rings = [
            (comm_r0, send_r0, recv_r0, nxt_, 0),
            (comm_l0, send_l0, recv_l0, prv_, nh),
            (comm_r1, send_r1, recv_r1, nxt_, nq),
            (comm_l1, send_l1, recv_l1, prv_, nh + nq),
        ]

        def mk(k, t):
            buf, ssem, rsem, tgt, _ = rings[k]
            return pltpu.make_async_remote_copy(
                src_ref=buf.at[t % N_SLOTS],
                dst_ref=buf.at[(t + 1) % N_SLOTS],
                send_sem=ssem.at[t % N_SLOTS],
                recv_sem=rsem.at[(t + 1) % N_SLOTS],
                device_id=(tgt,),
                device_id_type=pl.DeviceIdType.MESH,
            )

        blk_r = block_r(0)
        blk_l = block_l(0)
        comm_r0[0, :, :] = blk_r[:, :nq].astype(jnp.bfloat16)
        comm_r1[0, :, :] = blk_r[:, nq:].astype(jnp.bfloat16)
        comm_l0[0, :, :] = blk_l[:, :nq].astype(jnp.bfloat16)
        comm_l1[0, :, :] = blk_l[:, nq:].astype(jnp.bfloat16)
        rd = [mk(k, 0) for k in range(4)]
        for k in range(4):
            rd[k].start()

        for t in range(N_DEV - 1):
            r = (t + 1) % N_SLOTS
            blk_r = block_r(t + 1)
            blk_l = block_l(t + 1)
            halves = (blk_r[:, :nq], blk_l[:, :nq],
                      blk_r[:, nq:], blk_l[:, nq:])
            for k, blk in ((0, halves[0]), (1, halves[1]),
                           (2, halves[2]), (3, halves[3])):
                buf = rings[k][0]
                rd[k].wait()
                acc = buf[r, :, :].astype(jnp.float32) + blk
                if t < N_DEV - 2:
                    buf[r, :, :] = acc.astype(jnp.bfloat16)
                    rd[k] = mk(k, t + 1)
                    rd[k].start()
                else:
                    col = rings[k][4]
                    out_ref[:, col:col + nq] = jnp.maximum(acc, 0.0)

    return pl.pallas_call(
        body,
        out_shape=jax.ShapeDtypeStruct((m_per, n), jnp.float32),
        in_specs=[
            pl.BlockSpec(memory_space=pltpu.SMEM),
            pl.BlockSpec(memory_space=pltpu.SMEM),
            pl.BlockSpec(memory_space=pltpu.SMEM),
            pl.BlockSpec(memory_space=pltpu.VMEM),
            pl.BlockSpec(memory_space=pltpu.VMEM),
        ],
        out_specs=pl.BlockSpec(memory_space=pltpu.VMEM),
        scratch_shapes=[pltpu.VMEM((k_per, n), jnp.bfloat16)]
        + [pltpu.VMEM((N_SLOTS, m_per, nq), jnp.bfloat16)] * 4
        + [pltpu.SemaphoreType.DMA((N_SLOTS,))] * 8,
        compiler_params=pltpu.CompilerParams(collective_id=0),
    )(meta, sched_r, sched_l, x, w_mat)


# device time: 368269 ns/iter; 2.1497x vs baseline; 1.0004x over previous
import jax
import jax.numpy as jnp
from jax import lax
from jax.experimental import pallas as pl
from jax.experimental.pallas import tpu as pltpu

N_DEV = 32
N_SLOTS = 4
SUBS = 4

_PLANE_SNAKE = [(0, 0), (1, 0), (1, 1), (0, 1), (0, 2), (1, 2), (1, 3), (0, 3)]
_LOGICAL_OF = {}
for _z in range(4):
    for _xy in _PLANE_SNAKE:
        _LOGICAL_OF[(_xy[0], _xy[1], _z)] = len(_LOGICAL_OF)

_CYC_COORDS = []
for _z in range(4):
    _ys = range(4) if _z % 2 == 0 else range(3, -1, -1)
    _CYC_COORDS += [(0, _y, _z) for _y in _ys]
for _z in range(3, -1, -1):
    _ys = range(4) if _z % 2 == 1 else range(3, -1, -1)
    _CYC_COORDS += [(1, _y, _z) for _y in _ys]
assert len(set(_CYC_COORDS)) == N_DEV
for _a, _b in zip(_CYC_COORDS, _CYC_COORDS[1:] + _CYC_COORDS[:1]):
    assert sum(abs(_i - _j) for _i, _j in zip(_a, _b)) == 1, (_a, _b)

_CYCLE = [_LOGICAL_OF[c] for c in _CYC_COORDS]
_INV = [0] * N_DEV
for _p, _l in enumerate(_CYCLE):
    _INV[_l] = _p


def kernel(x, w_mat):
    m_global, k_per = x.shape
    _, n = w_mat.shape
    m_per = m_global // N_DEV
    nh = n // 2
    nn = nh // SUBS

    cyc = jnp.asarray(_CYCLE, dtype=jnp.int32)
    inv = jnp.asarray(_INV, dtype=jnp.int32)
    my = lax.axis_index("i")
    j = inv[my]
    nxt = cyc[(j + 1) % N_DEV]
    prv = cyc[(j - 1) % N_DEV]
    i_arr = jnp.arange(N_DEV, dtype=jnp.int32)
    sched_r = cyc[(j - 1 - i_arr) % N_DEV]
    sched_l = cyc[(j + 1 + i_arr) % N_DEV]
    meta = jnp.stack([nxt, prv]).astype(jnp.int32)

    def body(meta_ref, schr_ref, schl_ref, x_ref, w_ref, out_ref, wbf_ref,
             *comm_and_sems):
        bufs = comm_and_sems[: 2 * SUBS]
        sems = comm_and_sems[2 * SUBS:]
        nxt_ = meta_ref[0]
        prv_ = meta_ref[1]

        barrier = pltpu.get_barrier_semaphore()
        for nbr in (nxt_, prv_):
            pl.semaphore_signal(
                barrier, inc=1, device_id=(nbr,),
                device_id_type=pl.DeviceIdType.MESH,
            )
        pl.semaphore_wait(barrier, 2)

        wbf_ref[...] = w_ref[...].astype(jnp.bfloat16)

        def block_r(i):
            c = schr_ref[i]
            xa = x_ref[pl.ds(c * m_per, m_per), :].astype(jnp.bfloat16)
            return jnp.dot(xa, wbf_ref[:, :nh],
                           preferred_element_type=jnp.float32)

        def block_l(i):
            c = schl_ref[i]
            xa = x_ref[pl.ds(c * m_per, m_per), :].astype(jnp.bfloat16)
            return jnp.dot(xa, wbf_ref[:, nh:],
                           preferred_element_type=jnp.float32)

        def ring(k):
            tgt = nxt_ if k % 2 == 0 else prv_
            col = (k % 2) * nh + (k // 2) * nn
            return bufs[k], sems[2 * k], sems[2 * k + 1], tgt, col

        def mk(k, t):
            buf, ssem, rsem, tgt, _ = ring(k)
            return pltpu.make_async_remote_copy(
                src_ref=buf.at[t % N_SLOTS],
                dst_ref=buf.at[(t + 1) % N_SLOTS],
                send_sem=ssem.at[t % N_SLOTS],
                recv_sem=rsem.at[(t + 1) % N_SLOTS],
                device_id=(tgt,),
                device_id_type=pl.DeviceIdType.MESH,
            )

        def halves(blkr, blkl, k):
            blk = blkr if k % 2 == 0 else blkl
            c0 = (k // 2) * nn
            return blk[:, c0:c0 + nn]

        rd = [None] * (2 * SUBS)
        blk_r = block_r(0)
        blk_l = block_l(0)
        for k in range(2 * SUBS):
            ring(k)[0][0, :, :] = halves(blk_r, blk_l, k).astype(jnp.bfloat16)
            rd[k] = mk(k, 0)
            rd[k].start()

        for t in range(N_DEV - 1):
            r = (t + 1) % N_SLOTS
            blk_r = block_r(t + 1)
            blk_l = block_l(t + 1)
            for k in range(2 * SUBS):
                buf = ring(k)[0]
                rd[k].wait()
                acc = buf[r, :, :].astype(jnp.float32) + halves(blk_r, blk_l, k)
                if t < N_DEV - 2:
                    buf[r, :, :] = acc.astype(jnp.bfloat16)
                    rd[k] = mk(k, t + 1)
                    rd[k].start()
                else:
                    col = ring(k)[4]
                    out_ref[:, col:col + nn] = jnp.maximum(acc, 0.0)

    return pl.pallas_call(
        body,
        out_shape=jax.ShapeDtypeStruct((m_per, n), jnp.float32),
        in_specs=[
            pl.BlockSpec(memory_space=pltpu.SMEM),
            pl.BlockSpec(memory_space=pltpu.SMEM),
            pl.BlockSpec(memory_space=pltpu.SMEM),
            pl.BlockSpec(memory_space=pltpu.VMEM),
            pl.BlockSpec(memory_space=pltpu.VMEM),
        ],
        out_specs=pl.BlockSpec(memory_space=pltpu.VMEM),
        scratch_shapes=[pltpu.VMEM((k_per, n), jnp.bfloat16)]
        + [pltpu.VMEM((N_SLOTS, m_per, nn), jnp.bfloat16)] * (2 * SUBS)
        + [pltpu.SemaphoreType.DMA((N_SLOTS,))] * (4 * SUBS),
        compiler_params=pltpu.CompilerParams(collective_id=0),
    )(meta, sched_r, sched_l, x, w_mat)
